# baseline (speedup 1.0000x reference)
import functools

import jax
import jax.numpy as jnp
from jax import lax
from jax.experimental import pallas as pl
from jax.experimental.pallas import tpu as pltpu
from jax.experimental.pallas import tpu_sc as plsc

TOPK = 16
L = 16
NC = 2
NS = 16
NW = NC * NS
SUP = 256

_NEG_INF = float("-inf")


def _cmp_exchange(tv, ti, p, iota, f32_s, i32_s):
    f32_s[...] = tv
    i32_s[...] = ti
    pv = plsc.load_gather(f32_s, [p])
    pi = plsc.load_gather(i32_s, [p])
    left = iota < p
    win = (tv > pv) | ((tv == pv) & (ti < pi))
    take_self = (win == left) | (p == iota)
    tv = jnp.where(take_self, tv, pv)
    ti = jnp.where(take_self, ti, pi)
    return tv, ti


def _compress_step(ref, idx_vals, msk, basem1):
    cum = plsc.cumsum(msk.astype(jnp.int32))
    plsc.store_scatter(ref, [basem1 + cum], idx_vals, mask=msk)
    return basem1 + plsc.all_reduce_population_count(msk)


def _max_pass(buf, msup, n, iota):
    def sup_body(s, m_run):
        v = buf[pl.ds(s * SUP, L)]
        for j in range(1, SUP // L):
            v = jnp.maximum(v, buf[pl.ds(s * SUP + j * L, L)])
        msup[pl.ds(s * L, L)] = v
        return jnp.maximum(m_run, v)

    init = jnp.full((L,), _NEG_INF, jnp.float32)
    m_run = plsc.parallel_loop(0, n // SUP, unroll=2, carry=init)(sup_body)
    return jnp.broadcast_to(jnp.min(m_run), (L,))


def _hint_pass(msup, hint, n, thr0, iota):
    def sup_body(s, basem1):
        mv = msup[pl.ds(s * L, L)]
        return _compress_step(hint, iota + s * L, mv >= thr0, basem1)

    init = jnp.full((L,), -1, jnp.int32)
    basem1 = plsc.parallel_loop(0, n // SUP, unroll=2, carry=init)(sup_body)
    nhint = jnp.max(basem1) + 1
    plsc.store_scatter(hint, [nhint + iota], jnp.full((L,), (n // SUP) * L, jnp.int32))
    return nhint


def _drill(buf, hint, cand, nhint, thr0, iota):
    sub = lax.shift_left(iota, 4)

    def w_cond(c):
        k, _ = c
        return k < nhint

    def w_body(c):
        k, basem1 = c
        hvec = plsc.load_gather(hint, [jnp.broadcast_to(k, (L,))])
        eidx = (lax.shift_left(hvec & jnp.int32(-16), 4) | (hvec & 15)) + sub
        cv = plsc.load_gather(buf, [eidx])
        basem1 = _compress_step(cand, eidx, cv >= thr0, basem1)
        return k + 1, basem1

    init = (jnp.int32(0), jnp.full((L,), -1, jnp.int32))
    _, basem1 = lax.while_loop(w_cond, w_body, init)
    return jnp.max(basem1) + 1


def _merge_candidates(buf, cand, ncand, iota, f32_s, i32_s):
    p_even = lax.bitwise_xor(iota, jnp.int32(1))
    p_odd = jnp.clip(lax.bitwise_xor(iota - 1, jnp.int32(1)) + 1, 0, L - 1)
    stages = tuple(lax.bitwise_xor(iota, jnp.int32(d)) for d in (8, 4, 2, 1))

    def w_cond(c):
        i, _, _ = c
        return i < ncand

    def w_body(c):
        i, top_v, top_i = c
        valid = (iota + i) < ncand
        ci = jnp.where(valid, cand[pl.ds(i, L)], 0)
        cv = plsc.load_gather(buf, [ci])
        cv = jnp.where(valid, cv, _NEG_INF)
        cv, ci = plsc.sort_key_val(cv, ci, descending=True)
        for p in (p_even, p_odd, p_even, p_odd):
            cv, ci = _cmp_exchange(cv, ci, p, iota, f32_s, i32_s)
        rv = lax.rev(cv, (0,))
        ri = lax.rev(ci, (0,))
        take = (top_v > rv) | ((top_v == rv) & (top_i < ri))
        mv = jnp.where(take, top_v, rv)
        mi = jnp.where(take, top_i, ri)
        for p in stages:
            mv, mi = _cmp_exchange(mv, mi, p, iota, f32_s, i32_s)
        return i + L, mv, mi

    init = (jnp.int32(0), jnp.full((L,), _NEG_INF, jnp.float32), iota)
    _, top_v, top_i = lax.while_loop(w_cond, w_body, init)
    return top_v, top_i


def _build_sc_call(b, n):
    rows_per_w = b // NW
    n_hint_cap = (n // SUP) * L
    mesh = plsc.VectorSubcoreMesh(core_axis_name="c", subcore_axis_name="s")

    @functools.partial(
        pl.kernel,
        out_type=[
            jax.ShapeDtypeStruct((b * TOPK,), jnp.float32),
            jax.ShapeDtypeStruct((b * TOPK,), jnp.int32),
        ],
        mesh=mesh,
        compiler_params=pltpu.CompilerParams(needs_layout_passes=False),
        scratch_types=[
            pltpu.VMEM((n + SUP,), jnp.float32),
            pltpu.VMEM((n + SUP,), jnp.float32),
            pltpu.VMEM((n,), jnp.int32),
            pltpu.VMEM((n // SUP * L,), jnp.float32),
            pltpu.VMEM((n_hint_cap + L,), jnp.int32),
            pltpu.VMEM((TOPK,), jnp.float32),
            pltpu.VMEM((TOPK,), jnp.int32),
            pltpu.VMEM((TOPK,), jnp.int32),
            pltpu.SemaphoreType.DMA,
            pltpu.SemaphoreType.DMA,
            pltpu.SemaphoreType.DMA,
        ],
    )
    def sc_topk(x_hbm, labels_hbm, outv_hbm, outi_hbm,
                buf_a, buf_b, cand, msup, hint, f32_s, i32_s, lbl_s,
                sem_a, sem_b, sem_g):
        wid = lax.axis_index("s") * NC + lax.axis_index("c")
        base_row = wid * rows_per_w
        iota = lax.iota(jnp.int32, L)

        ninf = jnp.full((L,), _NEG_INF, jnp.float32)
        for j in range(SUP // L):
            buf_a[pl.ds(n + j * L, L)] = ninf
            buf_b[pl.ds(n + j * L, L)] = ninf

        bufs = (buf_a, buf_b)
        sems = (sem_a, sem_b)
        copies = [None] * rows_per_w
        copies[0] = pltpu.async_copy(x_hbm.at[base_row], buf_a.at[pl.ds(0, n)], sem_a)
        for r in range(rows_per_w):
            if r + 1 < rows_per_w:
                copies[r + 1] = pltpu.async_copy(
                    x_hbm.at[base_row + r + 1],
                    bufs[(r + 1) % 2].at[pl.ds(0, n)], sems[(r + 1) % 2])
            copies[r].wait()
            buf = bufs[r % 2]
            thr0 = _max_pass(buf, msup, n, iota)
            nhint = _hint_pass(msup, hint, n, thr0, iota)
            ncand = _drill(buf, hint, cand, nhint, thr0, iota)
            top_v, top_i = _merge_candidates(buf, cand, ncand, iota, f32_s, i32_s)
            i32_s[...] = top_i
            pltpu.async_copy(labels_hbm.at[i32_s], lbl_s, sem_g).wait()
            f32_s[...] = top_v
            out_off = (base_row + r) * TOPK
            pltpu.sync_copy(f32_s, outv_hbm.at[pl.ds(out_off, TOPK)])
            pltpu.sync_copy(lbl_s, outi_hbm.at[pl.ds(out_off, TOPK)])

    return sc_topk


def kernel(x, labels):
    b, n = x.shape
    out_v, out_l = _build_sc_call(b, n)(x, labels)
    return out_v.reshape(b, TOPK), out_l.reshape(b, TOPK)


# speedup vs baseline: 1.1113x; 1.1113x over previous
import functools

import jax
import jax.numpy as jnp
from jax import lax
from jax.experimental import pallas as pl
from jax.experimental.pallas import tpu as pltpu
from jax.experimental.pallas import tpu_sc as plsc

TOPK = 16
L = 16
NC = 2
NS = 16
NW = NC * NS
SUP = 128

_NEG_INF = float("-inf")


def _cmp_exchange(tv, ti, p, iota, f32_s, i32_s):
    f32_s[...] = tv
    i32_s[...] = ti
    pv = plsc.load_gather(f32_s, [p])
    pi = plsc.load_gather(i32_s, [p])
    left = iota < p
    win = (tv > pv) | ((tv == pv) & (ti < pi))
    take_self = (win == left) | (p == iota)
    tv = jnp.where(take_self, tv, pv)
    ti = jnp.where(take_self, ti, pi)
    return tv, ti


def _compress_step(ref, idx_vals, msk, basem1):
    cum = plsc.cumsum(msk.astype(jnp.int32))
    plsc.store_scatter(ref, [basem1 + cum], idx_vals, mask=msk)
    return basem1 + plsc.all_reduce_population_count(msk)


def _max_pass(buf, msup, n, iota):
    def sup_body(s, m_run):
        v = buf[pl.ds(s * SUP, L)]
        for j in range(1, SUP // L):
            v = jnp.maximum(v, buf[pl.ds(s * SUP + j * L, L)])
        msup[pl.ds(s * L, L)] = v
        return jnp.maximum(m_run, v)

    init = jnp.full((L,), _NEG_INF, jnp.float32)
    m_run = plsc.parallel_loop(0, n // SUP, unroll=4, carry=init)(sup_body)
    return jnp.broadcast_to(jnp.min(m_run), (L,))


def _hint_pass(msup, hint, n, thr0, iota):
    def sup_body(s, basem1):
        mv = msup[pl.ds(s * L, L)]
        return _compress_step(hint, iota + s * L, mv >= thr0, basem1)

    init = jnp.full((L,), -1, jnp.int32)
    basem1 = plsc.parallel_loop(0, n // SUP, unroll=2, carry=init)(sup_body)
    nhint = jnp.max(basem1) + 1
    plsc.store_scatter(hint, [nhint + iota], jnp.full((L,), (n // SUP) * L, jnp.int32))
    return nhint


def _drill(buf, hint, cand, nhint, thr0, iota):
    half = (iota >= 8).astype(jnp.int32)
    sub = lax.shift_left(iota & 7, 4)

    def w_cond(c):
        k, _ = c
        return k < nhint

    def w_body(c):
        k, basem1 = c
        hvec = plsc.load_gather(hint, [half + k])
        eidx = (lax.shift_left(hvec & jnp.int32(-16), 3) | (hvec & 15)) + sub
        cv = plsc.load_gather(buf, [eidx])
        basem1 = _compress_step(cand, eidx, cv >= thr0, basem1)
        return k + 2, basem1

    init = (jnp.int32(0), jnp.full((L,), -1, jnp.int32))
    _, basem1 = lax.while_loop(w_cond, w_body, init)
    return jnp.max(basem1) + 1


def _merge_candidates(buf, cand, ncand, iota, f32_s, i32_s):
    p_even = lax.bitwise_xor(iota, jnp.int32(1))
    p_odd = jnp.clip(lax.bitwise_xor(iota - 1, jnp.int32(1)) + 1, 0, L - 1)
    stages = tuple(lax.bitwise_xor(iota, jnp.int32(d)) for d in (8, 4, 2, 1))

    def w_cond(c):
        i, _, _ = c
        return i < ncand

    def w_body(c):
        i, top_v, top_i = c
        valid = (iota + i) < ncand
        ci = jnp.where(valid, cand[pl.ds(i, L)], 0)
        cv = plsc.load_gather(buf, [ci])
        cv = jnp.where(valid, cv, _NEG_INF)
        cv, ci = plsc.sort_key_val(cv, ci, descending=True)
        for p in (p_even, p_odd, p_even, p_odd):
            cv, ci = _cmp_exchange(cv, ci, p, iota, f32_s, i32_s)
        rv = lax.rev(cv, (0,))
        ri = lax.rev(ci, (0,))
        take = (top_v > rv) | ((top_v == rv) & (top_i < ri))
        mv = jnp.where(take, top_v, rv)
        mi = jnp.where(take, top_i, ri)
        for p in stages:
            mv, mi = _cmp_exchange(mv, mi, p, iota, f32_s, i32_s)
        return i + L, mv, mi

    init = (jnp.int32(0), jnp.full((L,), _NEG_INF, jnp.float32), iota)
    _, top_v, top_i = lax.while_loop(w_cond, w_body, init)
    return top_v, top_i


def _build_sc_call(b, n):
    rows_per_w = b // NW
    n_hint_cap = (n // SUP) * L
    mesh = plsc.VectorSubcoreMesh(core_axis_name="c", subcore_axis_name="s")

    @functools.partial(
        pl.kernel,
        out_type=[
            jax.ShapeDtypeStruct((b * TOPK,), jnp.float32),
            jax.ShapeDtypeStruct((b * TOPK,), jnp.int32),
        ],
        mesh=mesh,
        compiler_params=pltpu.CompilerParams(needs_layout_passes=False),
        scratch_types=[
            pltpu.VMEM((n + SUP,), jnp.float32),
            pltpu.VMEM((n + SUP,), jnp.float32),
            pltpu.VMEM((n,), jnp.int32),
            pltpu.VMEM((n // SUP * L,), jnp.float32),
            pltpu.VMEM((n_hint_cap + L,), jnp.int32),
            pltpu.VMEM((TOPK,), jnp.float32),
            pltpu.VMEM((TOPK,), jnp.int32),
            pltpu.VMEM((rows_per_w * TOPK,), jnp.float32),
            pltpu.VMEM((rows_per_w * TOPK,), jnp.int32),
            pltpu.VMEM((rows_per_w * TOPK,), jnp.int32),
            pltpu.SemaphoreType.DMA,
            pltpu.SemaphoreType.DMA,
            pltpu.SemaphoreType.DMA,
        ],
    )
    def sc_topk(x_hbm, labels_hbm, outv_hbm, outi_hbm,
                buf_a, buf_b, cand, msup, hint, f32_s, i32_s,
                outv_s, outi_s, lbl_s, sem_a, sem_b, sem_g):
        wid = lax.axis_index("s") * NC + lax.axis_index("c")
        base_row = wid * rows_per_w
        iota = lax.iota(jnp.int32, L)

        ninf = jnp.full((L,), _NEG_INF, jnp.float32)
        for j in range(SUP // L):
            buf_a[pl.ds(n + j * L, L)] = ninf
            buf_b[pl.ds(n + j * L, L)] = ninf

        bufs = (buf_a, buf_b)
        sems = (sem_a, sem_b)
        copies = [None] * rows_per_w
        copies[0] = pltpu.async_copy(x_hbm.at[base_row], buf_a.at[pl.ds(0, n)], sem_a)
        for r in range(rows_per_w):
            if r + 1 < rows_per_w:
                copies[r + 1] = pltpu.async_copy(
                    x_hbm.at[base_row + r + 1],
                    bufs[(r + 1) % 2].at[pl.ds(0, n)], sems[(r + 1) % 2])
            copies[r].wait()
            buf = bufs[r % 2]
            thr0 = _max_pass(buf, msup, n, iota)
            nhint = _hint_pass(msup, hint, n, thr0, iota)
            ncand = _drill(buf, hint, cand, nhint, thr0, iota)
            top_v, top_i = _merge_candidates(buf, cand, ncand, iota, f32_s, i32_s)
            outv_s[pl.ds(r * TOPK, TOPK)] = top_v
            outi_s[pl.ds(r * TOPK, TOPK)] = top_i
        pltpu.async_copy(labels_hbm.at[outi_s], lbl_s, sem_g).wait()
        out_off = base_row * TOPK
        pltpu.sync_copy(outv_s, outv_hbm.at[pl.ds(out_off, rows_per_w * TOPK)])
        pltpu.sync_copy(lbl_s, outi_hbm.at[pl.ds(out_off, rows_per_w * TOPK)])

    return sc_topk


def kernel(x, labels):
    b, n = x.shape
    out_v, out_l = _build_sc_call(b, n)(x, labels)
    return out_v.reshape(b, TOPK), out_l.reshape(b, TOPK)


# speedup vs baseline: 1.1340x; 1.0204x over previous
import functools

import jax
import jax.numpy as jnp
from jax import lax
from jax.experimental import pallas as pl
from jax.experimental.pallas import tpu as pltpu
from jax.experimental.pallas import tpu_sc as plsc

TOPK = 16
L = 16
NC = 2
NS = 16
NW = NC * NS
SUP = 128

_NEG_INF = float("-inf")


def _cmp_exchange(tv, ti, p, iota, f32_s, i32_s):
    f32_s[...] = tv
    i32_s[...] = ti
    pv = plsc.load_gather(f32_s, [p])
    pi = plsc.load_gather(i32_s, [p])
    left = iota < p
    win = (tv > pv) | ((tv == pv) & (ti < pi))
    take_self = (win == left) | (p == iota)
    tv = jnp.where(take_self, tv, pv)
    ti = jnp.where(take_self, ti, pi)
    return tv, ti


def _compress_step(ref, idx_vals, msk, basem1):
    cum = plsc.cumsum(msk.astype(jnp.int32))
    plsc.store_scatter(ref, [basem1 + cum], idx_vals, mask=msk)
    return basem1 + plsc.all_reduce_population_count(msk)


def _max_pass(buf, msup, n, iota):
    def sup_body(s, m_run):
        v = buf[pl.ds(s * SUP, L)]
        for j in range(1, SUP // L):
            v = jnp.maximum(v, buf[pl.ds(s * SUP + j * L, L)])
        msup[pl.ds(s * L, L)] = v
        return jnp.maximum(m_run, v)

    init = jnp.full((L,), _NEG_INF, jnp.float32)
    m_run = plsc.parallel_loop(0, n // SUP, unroll=4, carry=init)(sup_body)
    return jnp.broadcast_to(jnp.min(m_run), (L,))


def _hint_pass(msup, hint, n, thr0, iota):
    def sup_body(s, basem1):
        mv = msup[pl.ds(s * L, L)]
        return _compress_step(hint, iota + s * L, mv >= thr0, basem1)

    init = jnp.full((L,), -1, jnp.int32)
    basem1 = plsc.parallel_loop(0, n // SUP, unroll=4, carry=init)(sup_body)
    nhint = jnp.max(basem1) + 1
    plsc.store_scatter(hint, [nhint + iota], jnp.full((L,), (n // SUP) * L, jnp.int32))
    return nhint


def _drill(buf, hint, cand, nhint, thr0, iota):
    half = (iota >= 8).astype(jnp.int32)
    sub = lax.shift_left(iota & 7, 4)

    def w_cond(c):
        k, _ = c
        return k < nhint

    def w_body(c):
        k, basem1 = c
        hvec = plsc.load_gather(hint, [half + k])
        eidx = (lax.shift_left(hvec & jnp.int32(-16), 3) | (hvec & 15)) + sub
        cv = plsc.load_gather(buf, [eidx])
        basem1 = _compress_step(cand, eidx, cv >= thr0, basem1)
        return k + 2, basem1

    init = (jnp.int32(0), jnp.full((L,), -1, jnp.int32))
    _, basem1 = lax.while_loop(w_cond, w_body, init)
    return jnp.max(basem1) + 1


def _merge_candidates(buf, cand, ncand, iota, f32_s, i32_s):
    p_even = lax.bitwise_xor(iota, jnp.int32(1))
    p_odd = jnp.clip(lax.bitwise_xor(iota - 1, jnp.int32(1)) + 1, 0, L - 1)
    stages = tuple(lax.bitwise_xor(iota, jnp.int32(d)) for d in (8, 4, 2, 1))

    def w_cond(c):
        i, _, _ = c
        return i < ncand

    def w_body(c):
        i, top_v, top_i = c
        valid = (iota + i) < ncand
        ci = jnp.where(valid, cand[pl.ds(i, L)], 0)
        cv = plsc.load_gather(buf, [ci])
        cv = jnp.where(valid, cv, _NEG_INF)
        cv, ci = plsc.sort_key_val(cv, ci, descending=True)
        for p in (p_even, p_odd, p_even, p_odd):
            cv, ci = _cmp_exchange(cv, ci, p, iota, f32_s, i32_s)
        rv = lax.rev(cv, (0,))
        ri = lax.rev(ci, (0,))
        take = (top_v > rv) | ((top_v == rv) & (top_i < ri))
        mv = jnp.where(take, top_v, rv)
        mi = jnp.where(take, top_i, ri)
        for p in stages:
            mv, mi = _cmp_exchange(mv, mi, p, iota, f32_s, i32_s)
        return i + L, mv, mi

    init = (jnp.int32(0), jnp.full((L,), _NEG_INF, jnp.float32), iota)
    _, top_v, top_i = lax.while_loop(w_cond, w_body, init)
    return top_v, top_i


def _build_sc_call(b, n):
    rows_per_w = b // NW
    n_hint_cap = (n // SUP) * L
    mesh = plsc.VectorSubcoreMesh(core_axis_name="c", subcore_axis_name="s")

    @functools.partial(
        pl.kernel,
        out_type=[
            jax.ShapeDtypeStruct((b * TOPK,), jnp.float32),
            jax.ShapeDtypeStruct((b * TOPK,), jnp.int32),
        ],
        mesh=mesh,
        compiler_params=pltpu.CompilerParams(needs_layout_passes=False),
        scratch_types=[
            pltpu.VMEM((n + SUP,), jnp.float32),
            pltpu.VMEM((n + SUP,), jnp.float32),
            pltpu.VMEM((n,), jnp.int32),
            pltpu.VMEM((n // SUP * L,), jnp.float32),
            pltpu.VMEM((n_hint_cap + L,), jnp.int32),
            pltpu.VMEM((TOPK,), jnp.float32),
            pltpu.VMEM((TOPK,), jnp.int32),
            pltpu.VMEM((rows_per_w * TOPK,), jnp.float32),
            pltpu.VMEM((rows_per_w * TOPK,), jnp.int32),
            pltpu.VMEM((rows_per_w * TOPK,), jnp.int32),
            pltpu.SemaphoreType.DMA,
            pltpu.SemaphoreType.DMA,
            pltpu.SemaphoreType.DMA,
        ],
    )
    def sc_topk(x_hbm, labels_hbm, outv_hbm, outi_hbm,
                buf_a, buf_b, cand, msup, hint, f32_s, i32_s,
                outv_s, outi_s, lbl_s, sem_a, sem_b, sem_g):
        wid = lax.axis_index("s") * NC + lax.axis_index("c")
        base_row = wid * rows_per_w
        iota = lax.iota(jnp.int32, L)

        ninf = jnp.full((L,), _NEG_INF, jnp.float32)
        for j in range(SUP // L):
            buf_a[pl.ds(n + j * L, L)] = ninf
            buf_b[pl.ds(n + j * L, L)] = ninf

        bufs = (buf_a, buf_b)
        sems = (sem_a, sem_b)
        copies = [None] * rows_per_w
        copies[0] = pltpu.async_copy(x_hbm.at[base_row], buf_a.at[pl.ds(0, n)], sem_a)
        for r in range(rows_per_w):
            if r + 1 < rows_per_w:
                copies[r + 1] = pltpu.async_copy(
                    x_hbm.at[base_row + r + 1],
                    bufs[(r + 1) % 2].at[pl.ds(0, n)], sems[(r + 1) % 2])
            copies[r].wait()
            buf = bufs[r % 2]
            thr0 = _max_pass(buf, msup, n, iota)
            nhint = _hint_pass(msup, hint, n, thr0, iota)
            ncand = _drill(buf, hint, cand, nhint, thr0, iota)
            top_v, top_i = _merge_candidates(buf, cand, ncand, iota, f32_s, i32_s)
            outv_s[pl.ds(r * TOPK, TOPK)] = top_v
            outi_s[pl.ds(r * TOPK, TOPK)] = top_i
        pltpu.async_copy(labels_hbm.at[outi_s], lbl_s, sem_g).wait()
        out_off = base_row * TOPK
        pltpu.sync_copy(outv_s, outv_hbm.at[pl.ds(out_off, rows_per_w * TOPK)])
        pltpu.sync_copy(lbl_s, outi_hbm.at[pl.ds(out_off, rows_per_w * TOPK)])

    return sc_topk


def kernel(x, labels):
    b, n = x.shape
    out_v, out_l = _build_sc_call(b, n)(x, labels)
    return out_v.reshape(b, TOPK), out_l.reshape(b, TOPK)
